# baseline (speedup 1.0000x reference)
import functools

import jax
import jax.numpy as jnp
from jax import lax
from jax.experimental import pallas as pl
from jax.experimental.pallas import tpu as pltpu
from jax.experimental.pallas import tpu_sc as plsc

NUM_FIELDS = 26
VOCAB = 100000
EMBED = 32
BATCH = 4096
OUT_D = NUM_FIELDS * EMBED

_NC = 2
_NS = 16
_NW = _NC * _NS
_BPW = BATCH // _NW
_RH = _BPW // 2
_L = 16
_PACK = 128 // EMBED
_JU = 16

def _sc_embed(feats, tables):
    mesh = plsc.VectorSubcoreMesh(core_axis_name="c", subcore_axis_name="s")

    @functools.partial(
        pl.kernel,
        mesh=mesh,
        out_type=jax.ShapeDtypeStruct((BATCH, OUT_D), jnp.float32),
        scratch_types=[
            pltpu.VMEM((_RH,), jnp.int32),
            pltpu.VMEM((_RH,), jnp.int32),
            pltpu.VMEM((_RH, 128), jnp.float32),
            pltpu.VMEM((_RH, OUT_D), jnp.float32),
            pltpu.SemaphoreType.DMA,
        ],
        compiler_params=pltpu.CompilerParams(needs_layout_passes=False),
    )
    def k(*refs):
        fs = refs[:NUM_FIELDS]
        ws = refs[NUM_FIELDS:2 * NUM_FIELDS]
        out_hbm, idx_v, cidx_v, chunk_v, tile_v, sem = refs[2 * NUM_FIELDS:]
        wid = lax.axis_index("s") * _NC + lax.axis_index("c")
        base = wid * _BPW

        def do_field(i, rbase):
            pltpu.sync_copy(fs[i].at[pl.ds(rbase, _RH)], idx_v)

            def cdiv_body(g, _):
                v = idx_v[pl.ds(g * _L, _L)]
                cidx_v[pl.ds(g * _L, _L)] = jax.lax.shift_right_logical(v, 2)
                return ()
            jax.lax.fori_loop(0, _RH // _L, cdiv_body, (), unroll=True)
            pltpu.async_copy(ws[i].at[cidx_v], chunk_v, sem).wait()

            cb = i * EMBED

            def ext_body(g, _):
                rloc = jax.lax.iota(jnp.int32, _L) + g * _L
                offv = (idx_v[pl.ds(g * _L, _L)] & 3) * EMBED

                def j_body(jj, _):
                    for ju in range(_JU):
                        j = jj * _JU + ju
                        vals = plsc.load_gather(chunk_v, [rloc, offv + j])
                        plsc.store_scatter(
                            tile_v,
                            [rloc, jnp.full((_L,), cb, jnp.int32) + j],
                            vals)
                    return ()
                jax.lax.fori_loop(0, EMBED // _JU, j_body, ())
                return ()
            jax.lax.fori_loop(0, _RH // _L, ext_body, ())

        for h in range(_BPW // _RH):
            rbase = base + h * _RH
            for i in range(NUM_FIELDS):
                do_field(i, rbase)
            pltpu.sync_copy(tile_v, out_hbm.at[pl.ds(rbase, _RH), :])

    return k(*feats, *tables)


def kernel(feat_0, feat_1, feat_2, feat_3, feat_4, feat_5, feat_6, feat_7, feat_8, feat_9, feat_10, feat_11, feat_12, feat_13, feat_14, feat_15, feat_16, feat_17, feat_18, feat_19, feat_20, feat_21, feat_22, feat_23, feat_24, feat_25, W_0, W_1, W_2, W_3, W_4, W_5, W_6, W_7, W_8, W_9, W_10, W_11, W_12, W_13, W_14, W_15, W_16, W_17, W_18, W_19, W_20, W_21, W_22, W_23, W_24, W_25):
    feats = [feat_0, feat_1, feat_2, feat_3, feat_4, feat_5, feat_6, feat_7, feat_8, feat_9, feat_10, feat_11, feat_12, feat_13, feat_14, feat_15, feat_16, feat_17, feat_18, feat_19, feat_20, feat_21, feat_22, feat_23, feat_24, feat_25]
    tables = [W_0, W_1, W_2, W_3, W_4, W_5, W_6, W_7, W_8, W_9, W_10, W_11, W_12, W_13, W_14, W_15, W_16, W_17, W_18, W_19, W_20, W_21, W_22, W_23, W_24, W_25]
    tables2 = [w.reshape(VOCAB // _PACK, 128) for w in tables]
    return _sc_embed(feats, tables2)


# speedup vs baseline: 1.1387x; 1.1387x over previous
import functools

import jax
import jax.numpy as jnp
from jax import lax
from jax.experimental import pallas as pl
from jax.experimental.pallas import tpu as pltpu
from jax.experimental.pallas import tpu_sc as plsc

NUM_FIELDS = 26
VOCAB = 100000
EMBED = 32
BATCH = 4096
OUT_D = NUM_FIELDS * EMBED

_NC = 2
_NS = 16
_NW = _NC * _NS
_BPW = BATCH // _NW
_NH = 2
_RH = _BPW // _NH
_L = 16


def _sc_embed(feats, tables):
    mesh = plsc.VectorSubcoreMesh(core_axis_name="c", subcore_axis_name="s")

    @functools.partial(
        pl.kernel,
        mesh=mesh,
        out_type=jax.ShapeDtypeStruct((BATCH, OUT_D), jnp.float32),
        scratch_types=[
            pltpu.VMEM((_NH * NUM_FIELDS, _RH), jnp.int32),
            pltpu.VMEM((NUM_FIELDS, _RH, EMBED), jnp.float32),
            pltpu.VMEM((_RH, OUT_D), jnp.float32),
            pltpu.SemaphoreType.DMA,
            pltpu.SemaphoreType.DMA,
            pltpu.SemaphoreType.DMA,
        ],
        compiler_params=pltpu.CompilerParams(use_tc_tiling_on_sc=False),
    )
    def k(*refs):
        fs = refs[:NUM_FIELDS]
        ws = refs[NUM_FIELDS:2 * NUM_FIELDS]
        (out_hbm, idxs, rows, tile_v,
         sem_i, sem_g, sem_o) = refs[2 * NUM_FIELDS:]
        wid = lax.axis_index("s") * _NC + lax.axis_index("c")
        base = wid * _BPW

        idx_cps = []
        for h in range(_NH):
            for i in range(NUM_FIELDS):
                idx_cps.append(pltpu.async_copy(
                    fs[i].at[pl.ds(base + h * _RH, _RH)],
                    idxs.at[h * NUM_FIELDS + i], sem_i))
        for c in idx_cps:
            c.wait()

        for h in range(_NH):
            gcps = [pltpu.async_copy(
                        ws[i].at[idxs.at[h * NUM_FIELDS + i]],
                        rows.at[i], sem_g)
                    for i in range(NUM_FIELDS)]
            for c in gcps:
                c.wait()

            def row_body(r, _):
                for i in range(NUM_FIELDS):
                    for c in range(0, EMBED, _L):
                        tile_v[r, pl.ds(i * EMBED + c, _L)] = (
                            rows[i, r, pl.ds(c, _L)])
                return ()
            jax.lax.fori_loop(0, _RH, row_body, ())

            pltpu.sync_copy(
                tile_v, out_hbm.at[pl.ds(base + h * _RH, _RH), :])

    return k(*feats, *tables)


def kernel(feat_0, feat_1, feat_2, feat_3, feat_4, feat_5, feat_6, feat_7, feat_8, feat_9, feat_10, feat_11, feat_12, feat_13, feat_14, feat_15, feat_16, feat_17, feat_18, feat_19, feat_20, feat_21, feat_22, feat_23, feat_24, feat_25, W_0, W_1, W_2, W_3, W_4, W_5, W_6, W_7, W_8, W_9, W_10, W_11, W_12, W_13, W_14, W_15, W_16, W_17, W_18, W_19, W_20, W_21, W_22, W_23, W_24, W_25):
    feats = [feat_0, feat_1, feat_2, feat_3, feat_4, feat_5, feat_6, feat_7, feat_8, feat_9, feat_10, feat_11, feat_12, feat_13, feat_14, feat_15, feat_16, feat_17, feat_18, feat_19, feat_20, feat_21, feat_22, feat_23, feat_24, feat_25]
    tables = [W_0, W_1, W_2, W_3, W_4, W_5, W_6, W_7, W_8, W_9, W_10, W_11, W_12, W_13, W_14, W_15, W_16, W_17, W_18, W_19, W_20, W_21, W_22, W_23, W_24, W_25]
    return _sc_embed(feats, tables)
